# baseline (speedup 1.0000x reference)
import jax
import jax.numpy as jnp
from jax import lax
from jax.experimental import pallas as pl
from jax.experimental.pallas import tpu as pltpu
from jax.experimental.pallas import tpu_sc as plsc

N = 4096
D = 64
NNZ = 167772
NC = 2
NS = 16
NW = NC * NS
CHUNK = 128
NCHUNKS = 44
PER_TILE = NCHUNKS * CHUNK
NNZ_PAD = NW * PER_TILE
ROWS_PER_TILE = N // NS


def _sc_body(row_hbm, col_hbm, val_hbm, dense_hbm, out_hbm,
             row_v, col_v, val_v, in0, in1, in2, in3, out0, out1, out2, out3,
             zero_v, acc, dense_spm, gsem0, gsem1, gsem2, gsem3,
             ssem0, ssem1, ssem2, ssem3):
    cid = lax.axis_index("c")
    sid = lax.axis_index("s")
    wid = sid * NC + cid
    ins = (in0, in1, in2, in3)
    outs = (out0, out1, out2, out3)
    gsems = (gsem0, gsem1, gsem2, gsem3)
    ssems = (ssem0, ssem1, ssem2, ssem3)

    pltpu.make_async_copy(row_hbm.at[wid], row_v, gsem0).start()
    pltpu.make_async_copy(col_hbm.at[wid], col_v, gsem1).start()
    pltpu.make_async_copy(val_hbm.at[wid], val_v, gsem2).start()

    for r in range(16):
        for c in range(D // 16):
            zero_v[r, pl.ds(c * 16, 16)] = jnp.zeros((16,), jnp.float32)
    for z in range(ROWS_PER_TILE // 16):
        pltpu.make_async_copy(
            zero_v, acc.at[pl.ds(sid * ROWS_PER_TILE + z * 16, 16)],
            ssem0).start()
    pltpu.sync_copy(dense_hbm.at[pl.ds(sid * ROWS_PER_TILE, ROWS_PER_TILE)],
                    dense_spm.at[pl.ds(sid * ROWS_PER_TILE, ROWS_PER_TILE)])
    pltpu.make_async_copy(row_hbm.at[wid], row_v, gsem0).wait()
    pltpu.make_async_copy(col_hbm.at[wid], col_v, gsem1).wait()
    pltpu.make_async_copy(val_hbm.at[wid], val_v, gsem2).wait()
    for z in range(ROWS_PER_TILE // 16):
        pltpu.make_async_copy(
            zero_v, acc.at[pl.ds(sid * ROWS_PER_TILE + z * 16, 16)],
            ssem0).wait()
    plsc.subcore_barrier()

    def gather_start(j, b):
        pltpu.make_async_copy(
            dense_spm.at[col_v.at[j]], ins[b], gsems[b]).start()

    def gather_wait(j, b):
        pltpu.make_async_copy(
            dense_spm.at[col_v.at[j]], ins[b], gsems[b]).wait()

    def scatter_start(j, b):
        pltpu.make_async_copy(
            outs[b], acc.at[row_v.at[j]], ssems[b]).start(add=True)

    def scatter_wait(j, b):
        pltpu.make_async_copy(
            outs[b], acc.at[row_v.at[j]], ssems[b]).wait()

    def scale(j, b):
        in_v = ins[b]
        out_v = outs[b]
        jdx = jnp.full((16,), j, jnp.int32)

        def group_body(g, _):
            e0 = g * 4
            svals = [plsc.load_gather(
                val_v, [jdx, jnp.full((16,), e0 + q, jnp.int32)])
                for q in range(4)]
            rvs = [in_v[e0 + q, pl.ds(c * 16, 16)]
                   for q in range(4) for c in range(D // 16)]
            for q in range(4):
                for c in range(D // 16):
                    out_v[e0 + q, pl.ds(c * 16, 16)] = (
                        rvs[q * (D // 16) + c] * svals[q])
            return 0

        lax.fori_loop(0, CHUNK // 4, group_body, 0)

    def phase(j, b):
        gather_wait(j, b)

        @pl.when(j >= 4)
        def _():
            scatter_wait(j - 4, b)

        scale(j, b)
        scatter_start(j, b)

        @pl.when(j + 4 < NCHUNKS)
        def _():
            gather_start(j + 4, b)

    gather_start(0, 0)
    gather_start(1, 1)
    gather_start(2, 2)
    gather_start(3, 3)

    def quad_body(k, _):
        j = k * 4
        phase(j, 0)
        phase(j + 1, 1)
        phase(j + 2, 2)
        phase(j + 3, 3)
        return 0

    lax.fori_loop(0, NCHUNKS // 4, quad_body, 0)
    scatter_wait(NCHUNKS - 4, 0)
    scatter_wait(NCHUNKS - 3, 1)
    scatter_wait(NCHUNKS - 2, 2)
    scatter_wait(NCHUNKS - 1, 3)

    plsc.subcore_barrier()
    pltpu.sync_copy(acc.at[pl.ds(sid * ROWS_PER_TILE, ROWS_PER_TILE)],
                    out_hbm.at[cid, pl.ds(sid * ROWS_PER_TILE, ROWS_PER_TILE)])


def _add_partials_body(p_ref, o_ref):
    o_ref[...] = p_ref[0] + p_ref[1]


@jax.jit
def kernel(sparse_row, sparse_col, sparse_val, dense):
    pad = NNZ_PAD - NNZ
    row = jnp.concatenate(
        [sparse_row.astype(jnp.int32), jnp.zeros((pad,), jnp.int32)])
    col = jnp.concatenate(
        [sparse_col.astype(jnp.int32), jnp.zeros((pad,), jnp.int32)])
    val = jnp.concatenate(
        [sparse_val.astype(jnp.float32), jnp.zeros((pad,), jnp.float32)])
    row3 = row.reshape(NW, NCHUNKS, CHUNK)
    col3 = col.reshape(NW, NCHUNKS, CHUNK)
    val3 = val.reshape(NW, NCHUNKS, CHUNK)

    mesh = plsc.VectorSubcoreMesh(core_axis_name="c", subcore_axis_name="s")
    sc_call = pl.kernel(
        _sc_body,
        out_type=jax.ShapeDtypeStruct((NC, N, D), jnp.float32),
        mesh=mesh,
        compiler_params=pltpu.CompilerParams(
            needs_layout_passes=False, use_tc_tiling_on_sc=False),
        scratch_types=[
            pltpu.VMEM((NCHUNKS, CHUNK), jnp.int32),
            pltpu.VMEM((NCHUNKS, CHUNK), jnp.int32),
            pltpu.VMEM((NCHUNKS, CHUNK), jnp.float32),
            pltpu.VMEM((CHUNK, D), jnp.float32),
            pltpu.VMEM((CHUNK, D), jnp.float32),
            pltpu.VMEM((CHUNK, D), jnp.float32),
            pltpu.VMEM((CHUNK, D), jnp.float32),
            pltpu.VMEM((CHUNK, D), jnp.float32),
            pltpu.VMEM((CHUNK, D), jnp.float32),
            pltpu.VMEM((CHUNK, D), jnp.float32),
            pltpu.VMEM((CHUNK, D), jnp.float32),
            pltpu.VMEM((16, D), jnp.float32),
            pltpu.VMEM_SHARED((N, D), jnp.float32),
            pltpu.VMEM_SHARED((N, D), jnp.float32),
            pltpu.SemaphoreType.DMA,
            pltpu.SemaphoreType.DMA,
            pltpu.SemaphoreType.DMA,
            pltpu.SemaphoreType.DMA,
            pltpu.SemaphoreType.DMA,
            pltpu.SemaphoreType.DMA,
            pltpu.SemaphoreType.DMA,
            pltpu.SemaphoreType.DMA,
        ],
    )
    partial = sc_call(row3, col3, val3, dense.astype(jnp.float32))

    out = pl.pallas_call(
        _add_partials_body,
        out_shape=jax.ShapeDtypeStruct((N, D), jnp.float32),
    )(partial)
    return out


# speedup vs baseline: 1.0560x; 1.0560x over previous
import jax
import jax.numpy as jnp
from jax import lax
from jax.experimental import pallas as pl
from jax.experimental.pallas import tpu as pltpu
from jax.experimental.pallas import tpu_sc as plsc

N = 4096
D = 64
NNZ = 167772
NC = 2
NS = 16
NW = NC * NS
CHUNK = 128
NCHUNKS = 42
PER_TILE = NCHUNKS * CHUNK
NNZ_PAD = NW * PER_TILE
ROWS_PER_TILE = N // NS


def _sc_body(row_hbm, col_hbm, val_hbm, dense_hbm, out_hbm,
             row_v, col_v, val_v, in0, in1, in2, out0, out1, out2, zero_v,
             acc, dense_spm, gsem0, gsem1, gsem2, ssem0, ssem1, ssem2):
    cid = lax.axis_index("c")
    sid = lax.axis_index("s")
    wid = sid * NC + cid
    ins = (in0, in1, in2)
    outs = (out0, out1, out2)
    gsems = (gsem0, gsem1, gsem2)
    ssems = (ssem0, ssem1, ssem2)

    pltpu.make_async_copy(row_hbm.at[wid], row_v, gsem0).start()
    pltpu.make_async_copy(col_hbm.at[wid], col_v, gsem1).start()
    pltpu.make_async_copy(val_hbm.at[wid], val_v, gsem2).start()

    for r in range(16):
        for c in range(D // 16):
            zero_v[r, pl.ds(c * 16, 16)] = jnp.zeros((16,), jnp.float32)
    for z in range(ROWS_PER_TILE // 16):
        pltpu.make_async_copy(
            zero_v, acc.at[pl.ds(sid * ROWS_PER_TILE + z * 16, 16)],
            ssem0).start()
    pltpu.sync_copy(dense_hbm.at[pl.ds(sid * ROWS_PER_TILE, ROWS_PER_TILE)],
                    dense_spm.at[pl.ds(sid * ROWS_PER_TILE, ROWS_PER_TILE)])
    pltpu.make_async_copy(row_hbm.at[wid], row_v, gsem0).wait()
    pltpu.make_async_copy(col_hbm.at[wid], col_v, gsem1).wait()
    pltpu.make_async_copy(val_hbm.at[wid], val_v, gsem2).wait()
    for z in range(ROWS_PER_TILE // 16):
        pltpu.make_async_copy(
            zero_v, acc.at[pl.ds(sid * ROWS_PER_TILE + z * 16, 16)],
            ssem0).wait()
    plsc.subcore_barrier()

    def gather_start(j, b):
        pltpu.make_async_copy(
            dense_spm.at[col_v.at[j]], ins[b], gsems[b]).start()

    def gather_wait(j, b):
        pltpu.make_async_copy(
            dense_spm.at[col_v.at[j]], ins[b], gsems[b]).wait()

    def scatter_start(j, b):
        pltpu.make_async_copy(
            outs[b], acc.at[row_v.at[j]], ssems[b]).start(add=True)

    def scatter_wait(j, b):
        pltpu.make_async_copy(
            outs[b], acc.at[row_v.at[j]], ssems[b]).wait()

    def scale(j, b):
        in_v = ins[b]
        out_v = outs[b]
        jdx = jnp.full((16,), j, jnp.int32)

        def group_body(g, _):
            e0 = g * 4
            svals = [plsc.load_gather(
                val_v, [jdx, jnp.full((16,), e0 + q, jnp.int32)])
                for q in range(4)]
            rvs = [in_v[e0 + q, pl.ds(c * 16, 16)]
                   for q in range(4) for c in range(D // 16)]
            for q in range(4):
                for c in range(D // 16):
                    out_v[e0 + q, pl.ds(c * 16, 16)] = (
                        rvs[q * (D // 16) + c] * svals[q])
            return 0

        lax.fori_loop(0, CHUNK // 4, group_body, 0)

    def phase(j, b):
        gather_wait(j, b)

        @pl.when(j >= 3)
        def _():
            scatter_wait(j - 3, b)

        scale(j, b)
        scatter_start(j, b)

        @pl.when(j + 3 < NCHUNKS)
        def _():
            gather_start(j + 3, b)

    gather_start(0, 0)
    gather_start(1, 1)
    gather_start(2, 2)

    def tri_body(k, _):
        j = k * 3
        phase(j, 0)
        phase(j + 1, 1)
        phase(j + 2, 2)
        return 0

    lax.fori_loop(0, NCHUNKS // 3, tri_body, 0)
    scatter_wait(NCHUNKS - 3, 0)
    scatter_wait(NCHUNKS - 2, 1)
    scatter_wait(NCHUNKS - 1, 2)

    plsc.subcore_barrier()
    pltpu.sync_copy(acc.at[pl.ds(sid * ROWS_PER_TILE, ROWS_PER_TILE)],
                    out_hbm.at[cid, pl.ds(sid * ROWS_PER_TILE, ROWS_PER_TILE)])


def _add_partials_body(p_ref, o_ref):
    o_ref[...] = p_ref[0] + p_ref[1]


@jax.jit
def kernel(sparse_row, sparse_col, sparse_val, dense):
    pad = NNZ_PAD - NNZ
    row = jnp.concatenate(
        [sparse_row.astype(jnp.int32), jnp.zeros((pad,), jnp.int32)])
    col = jnp.concatenate(
        [sparse_col.astype(jnp.int32), jnp.zeros((pad,), jnp.int32)])
    val = jnp.concatenate(
        [sparse_val.astype(jnp.float32), jnp.zeros((pad,), jnp.float32)])
    row3 = row.reshape(NW, NCHUNKS, CHUNK)
    col3 = col.reshape(NW, NCHUNKS, CHUNK)
    val3 = val.reshape(NW, NCHUNKS, CHUNK)

    mesh = plsc.VectorSubcoreMesh(core_axis_name="c", subcore_axis_name="s")
    sc_call = pl.kernel(
        _sc_body,
        out_type=jax.ShapeDtypeStruct((NC, N, D), jnp.float32),
        mesh=mesh,
        compiler_params=pltpu.CompilerParams(
            needs_layout_passes=False, use_tc_tiling_on_sc=False),
        scratch_types=[
            pltpu.VMEM((NCHUNKS, CHUNK), jnp.int32),
            pltpu.VMEM((NCHUNKS, CHUNK), jnp.int32),
            pltpu.VMEM((NCHUNKS, CHUNK), jnp.float32),
            pltpu.VMEM((CHUNK, D), jnp.float32),
            pltpu.VMEM((CHUNK, D), jnp.float32),
            pltpu.VMEM((CHUNK, D), jnp.float32),
            pltpu.VMEM((CHUNK, D), jnp.float32),
            pltpu.VMEM((CHUNK, D), jnp.float32),
            pltpu.VMEM((CHUNK, D), jnp.float32),
            pltpu.VMEM((16, D), jnp.float32),
            pltpu.VMEM_SHARED((N, D), jnp.float32),
            pltpu.VMEM_SHARED((N, D), jnp.float32),
            pltpu.SemaphoreType.DMA,
            pltpu.SemaphoreType.DMA,
            pltpu.SemaphoreType.DMA,
            pltpu.SemaphoreType.DMA,
            pltpu.SemaphoreType.DMA,
            pltpu.SemaphoreType.DMA,
        ],
    )
    partial = sc_call(row3, col3, val3, dense.astype(jnp.float32))

    out = pl.pallas_call(
        _add_partials_body,
        out_shape=jax.ShapeDtypeStruct((N, D), jnp.float32),
    )(partial)
    return out
